# baseline (speedup 1.0000x reference)
import functools

import jax
import jax.numpy as jnp
from jax.experimental import pallas as pl
from jax.experimental.pallas import tpu as pltpu
from jax.experimental.pallas import tpu_sc as plsc

_F32 = jnp.float32
_HI = jax.lax.Precision.HIGHEST


def _knn_kernel(ctr_ref, asq_ref, bsq_ref, pT_ref, knn_ref, m_ref, *, n, bm, kk):
    iota = jax.lax.broadcasted_iota(jnp.int32, (bm, n), 1)
    inner = jax.lax.dot_general(ctr_ref[0], pT_ref[0], (((1,), (0,)), ((), ())))
    dist2 = jnp.clip((asq_ref[0] + bsq_ref[0]) - 2.0 * inner, 0.0, None)
    dist = jnp.sqrt(dist2 + 1e-8)
    m0 = jnp.where(dist <= 0.3, dist, 1e9)
    m_ref[...] = m0
    val = jnp.min(m0, axis=1, keepdims=True)
    cols = []
    big_i = jnp.int32(n)
    inf = jnp.float32(float("inf"))
    for _ in range(kk):
        m = m_ref[...]
        idx = jnp.min(jnp.where(m == val, iota, big_i), axis=1, keepdims=True)
        idx = jnp.minimum(idx, jnp.int32(n - 1))
        cols.append(idx)
        newm = jnp.where(iota == idx, inf, m)
        m_ref[...] = newm
        val = jnp.min(newm, axis=1, keepdims=True)
    knn_ref[0] = jnp.concatenate(cols, axis=1)


def _make_sc_gather(tot_rows, c):
    nw = 32
    per_w = tot_rows // nw
    nchunk = per_w // 128
    mesh = plsc.VectorSubcoreMesh(core_axis_name="c", subcore_axis_name="s")

    @functools.partial(
        pl.kernel, mesh=mesh,
        out_type=jax.ShapeDtypeStruct((tot_rows, c), _F32),
        scratch_types=[
            pltpu.VMEM((128,), jnp.int32),
            pltpu.VMEM((128, c), _F32),
            pltpu.SemaphoreType.DMA,
        ],
    )
    def gather(table_hbm, idx_hbm, out_hbm, idx_v, rows_v, sem):
        wid = jax.lax.axis_index("s") * 2 + jax.lax.axis_index("c")
        base = wid * per_w
        for t in range(nchunk):
            off = base + t * 128
            pltpu.sync_copy(idx_hbm.at[pl.ds(off, 128)], idx_v)
            pltpu.async_copy(table_hbm.at[idx_v], rows_v, sem).wait()
            pltpu.sync_copy(rows_v, out_hbm.at[pl.ds(off, 128)])

    return gather


def _mlp_kernel(nf_ref, cf_ref, s_ref, b_ref, w1_ref, b1_ref,
                w2_ref, b2_ref, w3_ref, b3_ref, o_ref, *, bm, c, kk):
    cf = cf_ref[0]
    sc_c = s_ref[0:1, 0:c]
    sc_n = s_ref[0:1, c:2 * c]
    sb_c = b_ref[0:1, 0:c]
    sb_n = b_ref[0:1, c:2 * c]
    w1a = w1_ref[0:c, :]
    w1b = w1_ref[c:2 * c, :]
    b1r = b1_ref[0:1, :]
    b2r = b2_ref[0:1, :]
    b3r = b3_ref[0:1, :]
    scf = jnp.sum(cf, axis=1, keepdims=True)
    scf2 = jnp.sum(cf * cf, axis=1, keepdims=True)
    acc = jnp.zeros((bm, c), dtype=_F32)
    denom = jnp.float32(2 * c)
    for k in range(kk):
        nf = nf_ref[0][:, k * c:(k + 1) * c]
        sn = jnp.sum(nf, axis=1, keepdims=True)
        sn2 = jnp.sum(nf * nf, axis=1, keepdims=True)
        mu = (scf + sn) / denom
        var = (scf2 + sn2) / denom - mu * mu
        inv = jax.lax.rsqrt(var + 1e-5)
        ncf = (cf - mu) * inv * sc_c + sb_c
        nnf = (nf - mu) * inv * sc_n + sb_n
        h = jnp.maximum(
            jnp.dot(ncf, w1a, preferred_element_type=_F32)
            + jnp.dot(nnf, w1b, preferred_element_type=_F32)
            + b1r, 0.0)
        cw = jax.nn.sigmoid(
            jnp.dot(h, w2_ref[...], preferred_element_type=_F32)
            + b2r)
        acc = acc + nf * cw
    weighted = acc * jnp.float32(1.0 / kk)
    fused = weighted + cf
    o_ref[0] = jnp.maximum(
        jnp.dot(fused, w3_ref[...], preferred_element_type=_F32)
        + b3r, 0.0)


def kernel(points, feats, center_idx, ln_scale, ln_bias, W1, b1, W2, b2, W3, b3):
    B, N, _ = points.shape
    _, M = center_idx.shape
    C = feats.shape[-1]
    OUT = W3.shape[0]
    K = 16
    BM = 128 if M % 128 == 0 else M
    NB = M // BM

    cidx = center_idx.astype(jnp.int32)
    points_T = jnp.swapaxes(points, 1, 2)
    centers = jnp.take_along_axis(points, center_idx[:, :, None], axis=1)
    a_sq = jnp.sum(centers ** 2, axis=-1, keepdims=True)
    b_sq = jnp.swapaxes(jnp.sum(points ** 2, axis=-1, keepdims=True), 1, 2)

    BMA = 256 if M % 256 == 0 else BM
    NBA = M // BMA
    knn = pl.pallas_call(
        functools.partial(_knn_kernel, n=N, bm=BMA, kk=K),
        grid=(B, NBA),
        in_specs=[
            pl.BlockSpec((1, BMA, 3), lambda b, m: (b * NBA + m, 0, 0)),
            pl.BlockSpec((1, BMA, 1), lambda b, m: (b * NBA + m, 0, 0)),
            pl.BlockSpec((1, 1, N), lambda b, m: (b, 0, 0)),
            pl.BlockSpec((1, 3, N), lambda b, m: (b, 0, 0)),
        ],
        out_specs=pl.BlockSpec((1, BMA, K), lambda b, m: (b, m, 0)),
        out_shape=jax.ShapeDtypeStruct((B, M, K), jnp.int32),
        scratch_shapes=[pltpu.VMEM((BMA, N), _F32)],
    )(centers.reshape(B * NBA, BMA, 3),
      a_sq.reshape(B * NBA, BMA, 1), b_sq, points_T)

    boff = (jnp.arange(B, dtype=jnp.int32) * N)
    gidx = jnp.concatenate([
        (knn + boff[:, None, None]).reshape(-1),
        (cidx + boff[:, None]).reshape(-1),
    ])
    tot = B * M * K + B * M
    gathered = _make_sc_gather(tot, C)(feats.reshape(B * N, C), gidx)
    nf_in = gathered[:B * M * K].reshape(B * NB, BM, K * C)
    cf_in = gathered[B * M * K:].reshape(B * NB, BM, C)

    out = pl.pallas_call(
        functools.partial(_mlp_kernel, bm=BM, c=C, kk=K),
        grid=(B, NB),
        in_specs=[
            pl.BlockSpec((1, BM, K * C), lambda b, m: (b * NB + m, 0, 0)),
            pl.BlockSpec((1, BM, C), lambda b, m: (b * NB + m, 0, 0)),
            pl.BlockSpec((1, 2 * C), lambda b, m: (0, 0)),
            pl.BlockSpec((1, 2 * C), lambda b, m: (0, 0)),
            pl.BlockSpec((2 * C, C), lambda b, m: (0, 0)),
            pl.BlockSpec((1, C), lambda b, m: (0, 0)),
            pl.BlockSpec((C, C), lambda b, m: (0, 0)),
            pl.BlockSpec((1, C), lambda b, m: (0, 0)),
            pl.BlockSpec((C, OUT), lambda b, m: (0, 0)),
            pl.BlockSpec((1, OUT), lambda b, m: (0, 0)),
        ],
        out_specs=pl.BlockSpec((1, BM, OUT), lambda b, m: (b, m, 0)),
        out_shape=jax.ShapeDtypeStruct((B, M, OUT), _F32),
    )(nf_in, cf_in,
      ln_scale.reshape(1, 2 * C), ln_bias.reshape(1, 2 * C),
      W1.T, b1.reshape(1, C), W2.T, b2.reshape(1, C),
      W3.T, b3.reshape(1, OUT))

    return (out, knn)


# speedup vs baseline: 1.1564x; 1.1564x over previous
import functools

import jax
import jax.numpy as jnp
from jax.experimental import pallas as pl
from jax.experimental.pallas import tpu as pltpu
from jax.experimental.pallas import tpu_sc as plsc

_F32 = jnp.float32
_HI = jax.lax.Precision.HIGHEST


def _knn_kernel(ctr_ref, asq_ref, bsq_ref, pT_ref, knn_ref, m_ref, *, n, bm, kk):
    iota = jax.lax.broadcasted_iota(jnp.int32, (bm, n), 1)
    inner = jax.lax.dot_general(ctr_ref[0], pT_ref[0], (((1,), (0,)), ((), ())))
    dist2 = jnp.clip((asq_ref[0] + bsq_ref[0]) - 2.0 * inner, 0.0, None)
    dist = jnp.sqrt(dist2 + 1e-8)
    m0 = jnp.where(dist <= 0.3, dist, 1e9)
    m_ref[...] = m0
    val = jnp.min(m0, axis=1, keepdims=True)
    cols = []
    big_i = jnp.int32(n)
    inf = jnp.float32(float("inf"))
    for _ in range(kk):
        m = m_ref[...]
        idx = jnp.min(jnp.where(m == val, iota, big_i), axis=1, keepdims=True)
        idx = jnp.minimum(idx, jnp.int32(n - 1))
        cols.append(idx)
        newm = jnp.where(iota == idx, inf, m)
        m_ref[...] = newm
        val = jnp.min(newm, axis=1, keepdims=True)
    knn_ref[0] = jnp.concatenate(cols, axis=1)


def _make_sc_gather(tot_rows, c):
    nw = 32
    per_w = tot_rows // nw
    nchunk = per_w // 128
    mesh = plsc.VectorSubcoreMesh(core_axis_name="c", subcore_axis_name="s")

    @functools.partial(
        pl.kernel, mesh=mesh,
        out_type=jax.ShapeDtypeStruct((tot_rows, c), _F32),
        scratch_types=[
            pltpu.VMEM((128,), jnp.int32),
            pltpu.VMEM((128, c), _F32),
            pltpu.SemaphoreType.DMA,
        ],
    )
    def gather(table_hbm, idx_hbm, out_hbm, idx_v, rows_v, sem):
        wid = jax.lax.axis_index("s") * 2 + jax.lax.axis_index("c")
        base = wid * per_w
        for t in range(nchunk):
            off = base + t * 128
            pltpu.sync_copy(idx_hbm.at[pl.ds(off, 128)], idx_v)
            pltpu.async_copy(table_hbm.at[idx_v], rows_v, sem).wait()
            pltpu.sync_copy(rows_v, out_hbm.at[pl.ds(off, 128)])

    return gather


def _mlp_kernel(nf_ref, cf_ref, s_ref, b_ref, w1_ref, b1_ref,
                w2_ref, b2_ref, w3_ref, b3_ref, o_ref, *, bm, c, kk):
    cf = cf_ref[0]
    sc_c = s_ref[0:1, 0:c]
    sc_n = s_ref[0:1, c:2 * c]
    sb_c = b_ref[0:1, 0:c]
    sb_n = b_ref[0:1, c:2 * c]
    w1a = w1_ref[0:c, :]
    w1b = w1_ref[c:2 * c, :]
    b1r = b1_ref[0:1, :]
    b2r = b2_ref[0:1, :]
    b3r = b3_ref[0:1, :]
    scf = jnp.sum(cf, axis=1, keepdims=True)
    scf2 = jnp.sum(cf * cf, axis=1, keepdims=True)
    acc = jnp.zeros((bm, c), dtype=_F32)
    denom = jnp.float32(2 * c)
    for k in range(kk):
        nf = nf_ref[0][:, k * c:(k + 1) * c]
        sn = jnp.sum(nf, axis=1, keepdims=True)
        sn2 = jnp.sum(nf * nf, axis=1, keepdims=True)
        mu = (scf + sn) / denom
        var = (scf2 + sn2) / denom - mu * mu
        inv = jax.lax.rsqrt(var + 1e-5)
        ncf = (cf - mu) * inv * sc_c + sb_c
        nnf = (nf - mu) * inv * sc_n + sb_n
        h = jnp.maximum(
            jnp.dot(ncf, w1a, preferred_element_type=_F32)
            + jnp.dot(nnf, w1b, preferred_element_type=_F32)
            + b1r, 0.0)
        cw = jax.nn.sigmoid(
            jnp.dot(h, w2_ref[...], preferred_element_type=_F32)
            + b2r)
        acc = acc + nf * cw
    weighted = acc * jnp.float32(1.0 / kk)
    fused = weighted + cf
    o_ref[0] = jnp.maximum(
        jnp.dot(fused, w3_ref[...], preferred_element_type=_F32)
        + b3r, 0.0)


def kernel(points, feats, center_idx, ln_scale, ln_bias, W1, b1, W2, b2, W3, b3):
    B, N, _ = points.shape
    _, M = center_idx.shape
    C = feats.shape[-1]
    OUT = W3.shape[0]
    K = 16
    BM = 128 if M % 128 == 0 else M
    NB = M // BM

    cidx = center_idx.astype(jnp.int32)
    points_T = jnp.swapaxes(points, 1, 2)
    centers = jnp.take_along_axis(points, center_idx[:, :, None], axis=1)
    a_sq = jnp.sum(centers ** 2, axis=-1, keepdims=True)
    b_sq = jnp.swapaxes(jnp.sum(points ** 2, axis=-1, keepdims=True), 1, 2)

    BMA = BM
    NBA = M // BMA
    knn = pl.pallas_call(
        functools.partial(_knn_kernel, n=N, bm=BMA, kk=K),
        grid=(B, NBA),
        in_specs=[
            pl.BlockSpec((1, BMA, 3), lambda b, m: (b * NBA + m, 0, 0)),
            pl.BlockSpec((1, BMA, 1), lambda b, m: (b * NBA + m, 0, 0)),
            pl.BlockSpec((1, 1, N), lambda b, m: (b, 0, 0)),
            pl.BlockSpec((1, 3, N), lambda b, m: (b, 0, 0)),
        ],
        out_specs=pl.BlockSpec((1, BMA, K), lambda b, m: (b, m, 0)),
        out_shape=jax.ShapeDtypeStruct((B, M, K), jnp.int32),
        scratch_shapes=[pltpu.VMEM((BMA, N), _F32)],
    )(centers.reshape(B * NBA, BMA, 3),
      a_sq.reshape(B * NBA, BMA, 1), b_sq, points_T)

    boff = (jnp.arange(B, dtype=jnp.int32) * N)
    gidx = jnp.concatenate([
        (knn + boff[:, None, None]).reshape(-1),
        (cidx + boff[:, None]).reshape(-1),
    ])
    tot = B * M * K + B * M
    gathered = _make_sc_gather(tot, C)(feats.reshape(B * N, C), gidx)
    nf_in = gathered[:B * M * K].reshape(B * NB, BM, K * C)
    cf_in = gathered[B * M * K:].reshape(B * NB, BM, C)

    out = pl.pallas_call(
        functools.partial(_mlp_kernel, bm=BM, c=C, kk=K),
        grid=(B, NB),
        in_specs=[
            pl.BlockSpec((1, BM, K * C), lambda b, m: (b * NB + m, 0, 0)),
            pl.BlockSpec((1, BM, C), lambda b, m: (b * NB + m, 0, 0)),
            pl.BlockSpec((1, 2 * C), lambda b, m: (0, 0)),
            pl.BlockSpec((1, 2 * C), lambda b, m: (0, 0)),
            pl.BlockSpec((2 * C, C), lambda b, m: (0, 0)),
            pl.BlockSpec((1, C), lambda b, m: (0, 0)),
            pl.BlockSpec((C, C), lambda b, m: (0, 0)),
            pl.BlockSpec((1, C), lambda b, m: (0, 0)),
            pl.BlockSpec((C, OUT), lambda b, m: (0, 0)),
            pl.BlockSpec((1, OUT), lambda b, m: (0, 0)),
        ],
        out_specs=pl.BlockSpec((1, BM, OUT), lambda b, m: (b, m, 0)),
        out_shape=jax.ShapeDtypeStruct((B, M, OUT), _F32),
    )(nf_in, cf_in,
      ln_scale.reshape(1, 2 * C), ln_bias.reshape(1, 2 * C),
      W1.T, b1.reshape(1, C), W2.T, b2.reshape(1, C),
      W3.T, b3.reshape(1, OUT))

    return (out, knn)
